# Initial kernel scaffold; baseline (speedup 1.0000x reference)
import functools

import jax
import jax.numpy as jnp
from jax.experimental import pallas as pl
from jax.experimental.pallas import tpu as pltpu

N = 10000
E = 160000
D = 256
_ROWS = 1000


def _dense_body(x_ref, wT_ref, b_ref, w1_ref, b1_ref, w2_ref, b2_ref,
                h_ref, a1_ref, a2_ref, m_ref):
    x = x_ref[...]
    h = jnp.dot(x, wT_ref[...], preferred_element_type=jnp.float32) + b_ref[...]
    h_ref[...] = h
    a1 = jnp.dot(h, w1_ref[...], preferred_element_type=jnp.float32) + b1_ref[0, 0]
    a2 = jnp.dot(h, w2_ref[...], preferred_element_type=jnp.float32) + b2_ref[0, 0]
    a1_ref[...] = a1
    a2_ref[...] = a2
    m1 = jnp.max(a1)
    m2 = jnp.max(a2)

    @pl.when(pl.program_id(0) == 0)
    def _():
        m_ref[0, 0] = m1
        m_ref[0, 1] = m2

    @pl.when(pl.program_id(0) != 0)
    def _():
        m_ref[0, 0] = jnp.maximum(m_ref[0, 0], m1)
        m_ref[0, 1] = jnp.maximum(m_ref[0, 1], m2)


def _dense_phase(features, W_layer, b_layer, W_a1, b_a1, W_a2, b_a2):
    grid = (N // _ROWS,)
    full = lambda i: (0, 0)
    row = lambda i: (i, 0)
    return pl.pallas_call(
        _dense_body,
        grid=grid,
        in_specs=[
            pl.BlockSpec((_ROWS, D), row),
            pl.BlockSpec((D, D), full),
            pl.BlockSpec((1, D), full),
            pl.BlockSpec((D, 1), full),
            pl.BlockSpec((1, 1), full),
            pl.BlockSpec((D, 1), full),
            pl.BlockSpec((1, 1), full),
        ],
        out_specs=[
            pl.BlockSpec((_ROWS, D), row),
            pl.BlockSpec((_ROWS, 1), row),
            pl.BlockSpec((_ROWS, 1), row),
            pl.BlockSpec((1, 2), full),
        ],
        out_shape=[
            jax.ShapeDtypeStruct((N, D), jnp.float32),
            jax.ShapeDtypeStruct((N, 1), jnp.float32),
            jax.ShapeDtypeStruct((N, 1), jnp.float32),
            jax.ShapeDtypeStruct((1, 2), jnp.float32),
        ],
    )(features, W_layer.T, b_layer[None, :], W_a1.T, b_a1[None, :],
      W_a2.T, b_a2[None, :])


def kernel(features, adj_indices, adj_values, W_layer, b_layer,
           W_a1, b_a1, W_a2, b_a2, w_a3, b_a3):
    h, a1, a2, m = _dense_phase(features, W_layer, b_layer, W_a1, b_a1,
                                W_a2, b_a2)
    a1 = a1[:, 0]
    a2 = a2[:, 0]
    shift = jax.nn.leaky_relu(m[0, 0] + m[0, 1], negative_slope=0.01)

    src = adj_indices[0, :]
    dst = adj_indices[1, :]
    v = jax.nn.leaky_relu(a1[src] + a2[dst], negative_slope=0.01)
    w = jnp.exp(v - shift)
    denom = jax.ops.segment_sum(w, src, num_segments=N)
    num = jax.ops.segment_sum(w[:, None] * h[dst], src, num_segments=N)
    return num / denom[:, None]


# speedup vs baseline: 1.3554x; 1.3554x over previous
import functools

import jax
import jax.numpy as jnp
from jax.experimental import pallas as pl
from jax.experimental.pallas import tpu as pltpu

N = 10000
E = 160000
D = 256
_ROWS = 1000


def _dense_body(x_ref, wT_ref, b_ref, w1_ref, b1_ref, w2_ref, b2_ref,
                h_ref, a1_ref, a2_ref, m_ref):
    x = x_ref[...]
    h = jnp.dot(x, wT_ref[...], preferred_element_type=jnp.float32) + b_ref[...]
    h_ref[...] = h
    a1 = jnp.dot(h, w1_ref[...], preferred_element_type=jnp.float32) + b1_ref[0, 0]
    a2 = jnp.dot(h, w2_ref[...], preferred_element_type=jnp.float32) + b2_ref[0, 0]
    a1_ref[...] = a1
    a2_ref[...] = a2
    m = jnp.concatenate(
        [jnp.full((1, 64), jnp.max(a1), jnp.float32),
         jnp.full((1, 64), jnp.max(a2), jnp.float32)], axis=1)

    @pl.when(pl.program_id(0) == 0)
    def _():
        m_ref[...] = m

    @pl.when(pl.program_id(0) != 0)
    def _():
        m_ref[...] = jnp.maximum(m_ref[...], m)


def _dense_phase(features, W_layer, b_layer, W_a1, b_a1, W_a2, b_a2):
    grid = (N // _ROWS,)
    full = lambda i: (0, 0)
    row = lambda i: (i, 0)
    return pl.pallas_call(
        _dense_body,
        grid=grid,
        in_specs=[
            pl.BlockSpec((_ROWS, D), row),
            pl.BlockSpec((D, D), full),
            pl.BlockSpec((1, D), full),
            pl.BlockSpec((D, 1), full),
            pl.BlockSpec((1, 1), full),
            pl.BlockSpec((D, 1), full),
            pl.BlockSpec((1, 1), full),
        ],
        out_specs=[
            pl.BlockSpec((_ROWS, D), row),
            pl.BlockSpec((_ROWS, 1), row),
            pl.BlockSpec((_ROWS, 1), row),
            pl.BlockSpec((1, 128), full),
        ],
        out_shape=[
            jax.ShapeDtypeStruct((N, D), jnp.float32),
            jax.ShapeDtypeStruct((N, 1), jnp.float32),
            jax.ShapeDtypeStruct((N, 1), jnp.float32),
            jax.ShapeDtypeStruct((1, 128), jnp.float32),
        ],
    )(features, W_layer.T, b_layer[None, :], W_a1.T, b_a1[None, :],
      W_a2.T, b_a2[None, :])


def kernel(features, adj_indices, adj_values, W_layer, b_layer,
           W_a1, b_a1, W_a2, b_a2, w_a3, b_a3):
    h, a1, a2, m = _dense_phase(features, W_layer, b_layer, W_a1, b_a1,
                                W_a2, b_a2)
    a1 = a1[:, 0]
    a2 = a2[:, 0]
    shift = jax.nn.leaky_relu(m[0, 0] + m[0, 64], negative_slope=0.01)

    src = adj_indices[0, :]
    dst = adj_indices[1, :]
    v = jax.nn.leaky_relu(a1[src] + a2[dst], negative_slope=0.01)
    w = jnp.exp(v - shift)
    denom = jax.ops.segment_sum(w, src, num_segments=N)
    num = jax.ops.segment_sum(w[:, None] * h[dst], src, num_segments=N)
    return num / denom[:, None]


# speedup vs baseline: 8.6120x; 6.3537x over previous
import functools

import jax
import jax.numpy as jnp
from jax import lax
from jax.experimental import pallas as pl
from jax.experimental.pallas import tpu as pltpu
from jax.experimental.pallas import tpu_sc as plsc

N = 10000
E = 160000
D = 256
_ROWS = 1000
_NSUB = 16
_EPW = E // _NSUB
_BLK = 2000
_G = 80
_NGRP = _BLK // _G
_NBLK = _EPW // _BLK
_ZOFF = 624
_ZROWS = 640


def _dense_body(x_ref, wT_ref, b_ref, w1_ref, b1_ref, w2_ref, b2_ref,
                h_ref, a1_ref, a2_ref, m_ref):
    x = x_ref[...]
    h = jnp.dot(x, wT_ref[...], preferred_element_type=jnp.float32) + b_ref[...]
    h_ref[...] = h
    a1 = jnp.dot(h, w1_ref[...], preferred_element_type=jnp.float32) + b1_ref[0, 0]
    a2 = jnp.dot(h, w2_ref[...], preferred_element_type=jnp.float32) + b2_ref[0, 0]
    a1_ref[...] = a1
    a2_ref[...] = a2
    m = jnp.concatenate(
        [jnp.full((1, 64), jnp.max(a1), jnp.float32),
         jnp.full((1, 64), jnp.max(a2), jnp.float32)], axis=1)

    @pl.when(pl.program_id(0) == 0)
    def _():
        m_ref[...] = m

    @pl.when(pl.program_id(0) != 0)
    def _():
        m_ref[...] = jnp.maximum(m_ref[...], m)


def _dense_phase(features, W_layer, b_layer, W_a1, b_a1, W_a2, b_a2):
    grid = (N // _ROWS,)
    full = lambda i: (0, 0)
    row = lambda i: (i, 0)
    return pl.pallas_call(
        _dense_body,
        grid=grid,
        in_specs=[
            pl.BlockSpec((_ROWS, D), row),
            pl.BlockSpec((D, D), full),
            pl.BlockSpec((1, D), full),
            pl.BlockSpec((D, 1), full),
            pl.BlockSpec((1, 1), full),
            pl.BlockSpec((D, 1), full),
            pl.BlockSpec((1, 1), full),
        ],
        out_specs=[
            pl.BlockSpec((_ROWS, D), row),
            pl.BlockSpec((_ROWS, 1), row),
            pl.BlockSpec((_ROWS, 1), row),
            pl.BlockSpec((1, 128), full),
        ],
        out_shape=[
            jax.ShapeDtypeStruct((N, D), jnp.float32),
            jax.ShapeDtypeStruct((N, 1), jnp.float32),
            jax.ShapeDtypeStruct((N, 1), jnp.float32),
            jax.ShapeDtypeStruct((1, 128), jnp.float32),
        ],
    )(features, W_layer.T, b_layer[None, :], W_a1.T, b_a1[None, :],
      W_a2.T, b_a2[None, :])


def _edge_body(src_hbm, gdx_hbm, a1_hbm, a2_hbm, shift_hbm, h2_hbm,
               num_hbm, den_hbm,
               shiftv, a1v, a2v, srcv, gdxv, wv, gidx, sidx, rows, denbuf,
               acc, dacc):
    c = lax.axis_index("c")
    s = lax.axis_index("s")
    ebase = s * _EPW

    @pl.loop(0, _G)
    def _(j):
        for k in range(8):
            rows[j, pl.ds(k * 16, 16)] = jnp.zeros((16,), jnp.float32)
        denbuf[j, pl.ds(0, 16)] = jnp.zeros((16,), jnp.float32)

    @pl.loop(0, 7)
    def _(t):
        pltpu.sync_copy(rows, acc.at[pl.ds(s * 625 + t * _G, _G)])
        pltpu.sync_copy(denbuf, dacc.at[pl.ds(s * 625 + t * _G, _G)])

    pltpu.sync_copy(rows.at[pl.ds(0, 65)], acc.at[pl.ds(s * 625 + 560, 65)])
    pltpu.sync_copy(denbuf.at[pl.ds(0, 65)], dacc.at[pl.ds(s * 625 + 560, 65)])

    pltpu.sync_copy(shift_hbm, shiftv)
    pltpu.sync_copy(a1_hbm, a1v)
    pltpu.sync_copy(a2_hbm, a2v)
    shift = shiftv[...]

    plsc.subcore_barrier()

    for blk in range(_NBLK):
        bbase = ebase + blk * _BLK
        pltpu.sync_copy(src_hbm.at[pl.ds(bbase, _BLK)], srcv)
        pltpu.sync_copy(gdx_hbm.at[c, pl.ds(bbase, _BLK)], gdxv)

        @pl.loop(0, _BLK, step=16)
        def _(i):
            sv = srcv[pl.ds(i, 16)]
            dv = lax.shift_right_logical(gdxv[pl.ds(i, 16)], 1)
            x = plsc.load_gather(a1v, [sv]) + plsc.load_gather(a2v, [dv])
            x = jnp.where(x >= 0, x, x * 0.01)
            wv[pl.ds(i, 16)] = jnp.exp(x - shift)

        @pl.loop(0, _NGRP)
        def _(g):
            goff = g * _G
            pltpu.sync_copy(src_hbm.at[pl.ds(bbase + goff, _G)], sidx)
            pltpu.sync_copy(gdx_hbm.at[c, pl.ds(bbase + goff, _G)], gidx)
            pltpu.sync_copy(h2_hbm.at[gidx], rows)

            @pl.loop(0, _G)
            def _(j):
                bw = plsc.load_gather(
                    wv, [jnp.full((16,), goff + j, jnp.int32)])
                for k in range(8):
                    rows[j, pl.ds(k * 16, 16)] = (
                        rows[j, pl.ds(k * 16, 16)] * bw)
                denbuf[j, pl.ds(0, 16)] = bw

            pltpu.sync_copy(rows, acc.at[sidx], add=True)
            pltpu.sync_copy(denbuf, dacc.at[sidx], add=True)

    plsc.subcore_barrier()
    pltpu.sync_copy(acc.at[pl.ds(s * 625, 625)],
                    num_hbm.at[c, pl.ds(s * 625, 625)])
    pltpu.sync_copy(dacc.at[pl.ds(s * 625, 625)],
                    den_hbm.at[c, pl.ds(s * 625, 625)])


def _edge_phase(src, gdx, a1, a2, shift16, h2):
    mesh = plsc.VectorSubcoreMesh(core_axis_name="c", subcore_axis_name="s",
                                  num_cores=2, num_subcores=_NSUB)
    return pl.kernel(
        _edge_body,
        out_type=[
            jax.ShapeDtypeStruct((2, N, 128), jnp.float32),
            jax.ShapeDtypeStruct((2, N, 16), jnp.float32),
        ],
        mesh=mesh,
        compiler_params=pltpu.CompilerParams(use_tc_tiling_on_sc=False,
                                             needs_layout_passes=False),
        scratch_types=[
            pltpu.VMEM((16,), jnp.float32),
            pltpu.VMEM((N,), jnp.float32),
            pltpu.VMEM((N,), jnp.float32),
            pltpu.VMEM((_BLK,), jnp.int32),
            pltpu.VMEM((_BLK,), jnp.int32),
            pltpu.VMEM((_BLK,), jnp.float32),
            pltpu.VMEM((_G,), jnp.int32),
            pltpu.VMEM((_G,), jnp.int32),
            pltpu.VMEM((_G, 128), jnp.float32),
            pltpu.VMEM((_G, 16), jnp.float32),
            pltpu.VMEM_SHARED((N, 128), jnp.float32),
            pltpu.VMEM_SHARED((N, 16), jnp.float32),
        ],
    )(src, gdx, a1, a2, shift16, h2)


def _div_body(num_ref, den_ref, o_ref):
    n0 = num_ref[0]
    n1 = num_ref[1]
    d0 = den_ref[0][:, 0:1]
    d1 = den_ref[1][:, 0:1]
    d0 = jnp.where(d0 > 0, d0, 1.0)
    d1 = jnp.where(d1 > 0, d1, 1.0)
    o_ref[...] = jnp.concatenate([n0 / d0, n1 / d1], axis=1)


def _div_phase(num, den):
    return pl.pallas_call(
        _div_body,
        grid=(N // _ROWS,),
        in_specs=[
            pl.BlockSpec((2, _ROWS, 128), lambda i: (0, i, 0)),
            pl.BlockSpec((2, _ROWS, 16), lambda i: (0, i, 0)),
        ],
        out_specs=pl.BlockSpec((_ROWS, D), lambda i: (i, 0)),
        out_shape=jax.ShapeDtypeStruct((N, D), jnp.float32),
    )(num, den)


def kernel(features, adj_indices, adj_values, W_layer, b_layer,
           W_a1, b_a1, W_a2, b_a2, w_a3, b_a3):
    h, a1, a2, m = _dense_phase(features, W_layer, b_layer, W_a1, b_a1,
                                W_a2, b_a2)
    shift = jax.nn.leaky_relu(m[0, 0] + m[0, 64], negative_slope=0.01)
    h2 = h.reshape(2 * N, 128)
    dst = adj_indices[1]
    gdx = jnp.stack([dst * 2, dst * 2 + 1])
    num, den = _edge_phase(adj_indices[0], gdx,
                           a1.reshape(N), a2.reshape(N),
                           jnp.full((16,), shift, jnp.float32), h2)
    return _div_phase(num, den)


# speedup vs baseline: 10.8536x; 1.2603x over previous
import functools

import jax
import jax.numpy as jnp
from jax import lax
from jax.experimental import pallas as pl
from jax.experimental.pallas import tpu as pltpu
from jax.experimental.pallas import tpu_sc as plsc

N = 10000
E = 160000
D = 256
_ROWS = 1000
_NSUB = 16
_EPW = E // _NSUB
_BLK = 2000
_G = 40
_NGRP = _BLK // _G
_NBLK = _EPW // _BLK
_ZOFF = 624
_ZROWS = 640


def _dense_body(x_ref, wT_ref, b_ref, w1_ref, b1_ref, w2_ref, b2_ref,
                h_ref, a1_ref, a2_ref, m_ref):
    x = x_ref[...]
    h = jnp.dot(x, wT_ref[...], preferred_element_type=jnp.float32) + b_ref[...]
    h_ref[...] = h
    a1 = jnp.dot(h, w1_ref[...], preferred_element_type=jnp.float32) + b1_ref[0, 0]
    a2 = jnp.dot(h, w2_ref[...], preferred_element_type=jnp.float32) + b2_ref[0, 0]
    a1_ref[...] = a1
    a2_ref[...] = a2
    m = jnp.concatenate(
        [jnp.full((1, 64), jnp.max(a1), jnp.float32),
         jnp.full((1, 64), jnp.max(a2), jnp.float32)], axis=1)

    @pl.when(pl.program_id(0) == 0)
    def _():
        m_ref[...] = m

    @pl.when(pl.program_id(0) != 0)
    def _():
        m_ref[...] = jnp.maximum(m_ref[...], m)


def _dense_phase(features, W_layer, b_layer, W_a1, b_a1, W_a2, b_a2):
    grid = (N // _ROWS,)
    full = lambda i: (0, 0)
    row = lambda i: (i, 0)
    return pl.pallas_call(
        _dense_body,
        grid=grid,
        in_specs=[
            pl.BlockSpec((_ROWS, D), row),
            pl.BlockSpec((D, D), full),
            pl.BlockSpec((1, D), full),
            pl.BlockSpec((D, 1), full),
            pl.BlockSpec((1, 1), full),
            pl.BlockSpec((D, 1), full),
            pl.BlockSpec((1, 1), full),
        ],
        out_specs=[
            pl.BlockSpec((_ROWS, D), row),
            pl.BlockSpec((_ROWS, 1), row),
            pl.BlockSpec((_ROWS, 1), row),
            pl.BlockSpec((1, 128), full),
        ],
        out_shape=[
            jax.ShapeDtypeStruct((N, D), jnp.float32),
            jax.ShapeDtypeStruct((N, 1), jnp.float32),
            jax.ShapeDtypeStruct((N, 1), jnp.float32),
            jax.ShapeDtypeStruct((1, 128), jnp.float32),
        ],
    )(features, W_layer.T, b_layer[None, :], W_a1.T, b_a1[None, :],
      W_a2.T, b_a2[None, :])


def _edge_body(src_hbm, gdx_hbm, a1_hbm, a2_hbm, shift_hbm, h2_hbm,
               num_hbm, den_hbm,
               shiftv, a1v, a2v, srcv, gdxv, wv, sidx, rows0, rows1,
               denbuf, gsem0, gsem1, acc, dacc):
    c = lax.axis_index("c")
    s = lax.axis_index("s")
    ebase = s * _EPW
    rows = (rows0, rows1)
    gsem = (gsem0, gsem1)

    @pl.loop(0, _G)
    def _(j):
        for k in range(8):
            rows0[j, pl.ds(k * 16, 16)] = jnp.zeros((16,), jnp.float32)
        denbuf[j, pl.ds(0, 16)] = jnp.zeros((16,), jnp.float32)

    @pl.loop(0, 15)
    def _(t):
        pltpu.sync_copy(rows0, acc.at[pl.ds(s * 625 + t * _G, _G)])
        pltpu.sync_copy(denbuf, dacc.at[pl.ds(s * 625 + t * _G, _G)])

    pltpu.sync_copy(rows0.at[pl.ds(0, 25)], acc.at[pl.ds(s * 625 + 600, 25)])
    pltpu.sync_copy(denbuf.at[pl.ds(0, 25)], dacc.at[pl.ds(s * 625 + 600, 25)])

    pltpu.sync_copy(shift_hbm, shiftv)
    pltpu.sync_copy(a1_hbm, a1v)
    pltpu.sync_copy(a2_hbm, a2v)
    shift = shiftv[...]

    plsc.subcore_barrier()

    for blk in range(_NBLK):
        bbase = ebase + blk * _BLK
        pltpu.sync_copy(src_hbm.at[pl.ds(bbase, _BLK)], srcv)
        pltpu.sync_copy(gdx_hbm.at[c, pl.ds(bbase, _BLK)], gdxv)

        @pl.loop(0, _BLK, step=16)
        def _(i):
            sv = srcv[pl.ds(i, 16)]
            dv = lax.shift_right_logical(gdxv[pl.ds(i, 16)], 1)
            x = plsc.load_gather(a1v, [sv]) + plsc.load_gather(a2v, [dv])
            x = jnp.where(x >= 0, x, x * 0.01)
            wv[pl.ds(i, 16)] = jnp.exp(x - shift)

        def _work(g, p, prefetch):
            pltpu.make_async_copy(
                h2_hbm.at[gdxv.at[pl.ds(0, _G)]], rows[p], gsem[p]).wait()
            if prefetch:
                pltpu.async_copy(
                    h2_hbm.at[gdxv.at[pl.ds((g + 1) * _G, _G)]],
                    rows[1 - p], gsem[1 - p])
            goff = g * _G

            @pl.loop(0, _G)
            def _(j):
                bw = plsc.load_gather(
                    wv, [jnp.full((16,), goff + j, jnp.int32)])
                for k in range(8):
                    rows[p][j, pl.ds(k * 16, 16)] = (
                        rows[p][j, pl.ds(k * 16, 16)] * bw)
                denbuf[j, pl.ds(0, 16)] = bw

            pltpu.sync_copy(src_hbm.at[pl.ds(bbase + goff, _G)], sidx)
            pltpu.sync_copy(rows[p], acc.at[sidx], add=True)
            pltpu.sync_copy(denbuf, dacc.at[sidx], add=True)

        pltpu.async_copy(h2_hbm.at[gdxv.at[pl.ds(0, _G)]], rows0, gsem0)

        @pl.loop(0, _NGRP - 2, step=2)
        def _(gg):
            _work(gg, 0, True)
            _work(gg + 1, 1, True)

        _work(_NGRP - 2, 0, True)
        _work(_NGRP - 1, 1, False)

    plsc.subcore_barrier()
    pltpu.sync_copy(acc.at[pl.ds(s * 625, 625)],
                    num_hbm.at[c, pl.ds(s * 625, 625)])
    pltpu.sync_copy(dacc.at[pl.ds(s * 625, 625)],
                    den_hbm.at[c, pl.ds(s * 625, 625)])


def _edge_phase(src, gdx, a1, a2, shift16, h2):
    mesh = plsc.VectorSubcoreMesh(core_axis_name="c", subcore_axis_name="s",
                                  num_cores=2, num_subcores=_NSUB)
    return pl.kernel(
        _edge_body,
        out_type=[
            jax.ShapeDtypeStruct((2, N, 128), jnp.float32),
            jax.ShapeDtypeStruct((2, N, 16), jnp.float32),
        ],
        mesh=mesh,
        compiler_params=pltpu.CompilerParams(use_tc_tiling_on_sc=False,
                                             needs_layout_passes=False),
        scratch_types=[
            pltpu.VMEM((16,), jnp.float32),
            pltpu.VMEM((N,), jnp.float32),
            pltpu.VMEM((N,), jnp.float32),
            pltpu.VMEM((_BLK,), jnp.int32),
            pltpu.VMEM((_BLK,), jnp.int32),
            pltpu.VMEM((_BLK,), jnp.float32),
            pltpu.VMEM((_G,), jnp.int32),
            pltpu.VMEM((_G, 128), jnp.float32),
            pltpu.VMEM((_G, 128), jnp.float32),
            pltpu.VMEM((_G, 16), jnp.float32),
            pltpu.SemaphoreType.DMA,
            pltpu.SemaphoreType.DMA,
            pltpu.VMEM_SHARED((N, 128), jnp.float32),
            pltpu.VMEM_SHARED((N, 16), jnp.float32),
        ],
    )(src, gdx, a1, a2, shift16, h2)


def _div_body(num_ref, den_ref, o_ref):
    n0 = num_ref[0]
    n1 = num_ref[1]
    d0 = den_ref[0][:, 0:1]
    d1 = den_ref[1][:, 0:1]
    d0 = jnp.where(d0 > 0, d0, 1.0)
    d1 = jnp.where(d1 > 0, d1, 1.0)
    o_ref[...] = jnp.concatenate([n0 / d0, n1 / d1], axis=1)


def _div_phase(num, den):
    return pl.pallas_call(
        _div_body,
        grid=(N // _ROWS,),
        in_specs=[
            pl.BlockSpec((2, _ROWS, 128), lambda i: (0, i, 0)),
            pl.BlockSpec((2, _ROWS, 16), lambda i: (0, i, 0)),
        ],
        out_specs=pl.BlockSpec((_ROWS, D), lambda i: (i, 0)),
        out_shape=jax.ShapeDtypeStruct((N, D), jnp.float32),
    )(num, den)


def kernel(features, adj_indices, adj_values, W_layer, b_layer,
           W_a1, b_a1, W_a2, b_a2, w_a3, b_a3):
    h, a1, a2, m = _dense_phase(features, W_layer, b_layer, W_a1, b_a1,
                                W_a2, b_a2)
    shift = jax.nn.leaky_relu(m[0, 0] + m[0, 64], negative_slope=0.01)
    h2 = h.reshape(2 * N, 128)
    dst = adj_indices[1]
    gdx = jnp.stack([dst * 2, dst * 2 + 1])
    num, den = _edge_phase(adj_indices[0], gdx,
                           a1.reshape(N), a2.reshape(N),
                           jnp.full((16,), shift, jnp.float32), h2)
    return _div_phase(num, den)
